# baseline (speedup 1.0000x reference)
import jax
import jax.numpy as jnp
from jax import lax
from jax.experimental import pallas as pl
from jax.experimental.pallas import tpu as pltpu
from jax.experimental.pallas import tpu_sc as plsc

_N = 10000
_E = 320000
_D = 128
_H = 4
_NEG = 0.2
_NC = 2
_NS = 16
_NW = _NC * _NS
_EPW = _E // _NW
_CH = 16
_NCH = _EPW // _CH
_DEPTH = 4
_AW = 144
_RPT = _N // _NS
_ZR = 25
_NPK = (_NCH + 1) // 2


def _proj_body(x_ref, wl_ref, wr_ref, xl_ref, xr_ref):
    xv = x_ref[...]
    xl_ref[...] = jnp.dot(xv, wl_ref[...], preferred_element_type=jnp.float32)
    xr_ref[...] = jnp.dot(xv, wr_ref[...], preferred_element_type=jnp.float32)


def _project(x, W_l, W_r):
    blk = 1000
    return pl.pallas_call(
        _proj_body,
        grid=(_N // blk,),
        in_specs=[
            pl.BlockSpec((blk, _D), lambda i: (i, 0)),
            pl.BlockSpec((_D, _D), lambda i: (0, 0)),
            pl.BlockSpec((_D, _D), lambda i: (0, 0)),
        ],
        out_specs=[
            pl.BlockSpec((blk, _D), lambda i: (i, 0)),
            pl.BlockSpec((blk, _D), lambda i: (i, 0)),
        ],
        out_shape=[jax.ShapeDtypeStruct((_N, _D), jnp.float32)] * 2,
    )(x, W_l, W_r)


def _edge_body(psrc_hbm, pdst_hbm, xl_hbm, xr_hbm, att_hbm, out_hbm,
               psrcv, pdstv, gsrc, gdst, sdst,
               rl0, rl1, rl2, rl3, rr0, rr1, rr2, rr3,
               buf0, buf1, buf2, buf3, attv, zbuf, acc,
               sl0, sl1, sl2, sl3, sr0, sr1, sr2, sr3,
               ss0, ss1, ss2, ss3):
    c = lax.axis_index("c")
    s = lax.axis_index("s")
    wid = c * _NS + s

    pltpu.sync_copy(att_hbm, attv)

    pltpu.sync_copy(psrc_hbm.at[wid], psrcv)
    pltpu.sync_copy(pdst_hbm.at[wid], pdstv)

    zero16 = jnp.zeros((16,), jnp.float32)

    def zrow(r, carry):
        for cc in range(_AW // 16):
            zbuf[r, pl.ds(cc * 16, 16)] = zero16
        return carry

    lax.fori_loop(0, _ZR, zrow, 0)
    for b in range(_RPT // _ZR):
        pltpu.sync_copy(zbuf, acc.at[pl.ds(s * _RPT + b * _ZR, _ZR)])
    plsc.subcore_barrier()

    att_k = [attv[pl.ds(k * 16, 16)] for k in range(8)]
    iota16 = lax.iota(jnp.int32, 16)
    masks = [iota16 == h for h in range(_H - 1)]

    rl = (rl0, rl1, rl2, rl3)
    rr = (rr0, rr1, rr2, rr3)
    buf = (buf0, buf1, buf2, buf3)
    sls = (sl0, sl1, sl2, sl3)
    srs = (sr0, sr1, sr2, sr3)
    sss = (ss0, ss1, ss2, ss3)

    def unpack_idx(j, slot):
        m = j // 2
        sel_hi = (j % 2) == 1
        ws = psrcv[m]
        wd = pdstv[m]
        lo_s, hi_s = ws & 0xFFFF, lax.shift_right_logical(ws, 16)
        lo_d, hi_d = wd & 0xFFFF, lax.shift_right_logical(wd, 16)
        gsrc[slot] = jnp.where(sel_hi, hi_s, lo_s)
        gdst[slot] = jnp.where(sel_hi, hi_d, lo_d)

    def issue(slot):
        pltpu.async_copy(xl_hbm.at[gsrc.at[slot]], rl[slot], sls[slot])
        pltpu.async_copy(xr_hbm.at[gdst.at[slot]], rr[slot], srs[slot])

    def wait_gather(slot):
        pltpu.make_async_copy(xl_hbm.at[gsrc.at[0]], rl[slot], sls[slot]).wait()
        pltpu.make_async_copy(xr_hbm.at[gdst.at[0]], rr[slot], srs[slot]).wait()

    def compute_chunk(slot):
        rls, rrs, bufs = rl[slot], rr[slot], buf[slot]
        sdst[slot] = gdst[slot]

        @plsc.parallel_loop(0, _CH, unroll=4)
        def edge(e):
            a = [rls[e, pl.ds(k * 16, 16)] for k in range(8)]
            t = []
            for k in range(8):
                sv = a[k] + rrs[e, pl.ds(k * 16, 16)]
                v = jnp.maximum(sv, _NEG * sv)
                t.append(v * att_k[k])
            pv = []
            for h in range(_H):
                r_h = jnp.sum(t[2 * h] + t[2 * h + 1])
                pv.append(jnp.exp(jnp.broadcast_to(r_h, (16,))))
            p_pack = jnp.where(masks[0], pv[0],
                               jnp.where(masks[1], pv[1],
                                         jnp.where(masks[2], pv[2], pv[3])))
            bufs[e, pl.ds(128, 16)] = p_pack
            for k in range(8):
                bufs[e, pl.ds(k * 16, 16)] = a[k] * pv[k // 2]

    def scatter(slot):
        pltpu.async_copy(buf[slot], acc.at[sdst.at[slot]], sss[slot], add=True)

    def wait_scatter(slot):
        pltpu.make_async_copy(buf[slot], acc.at[sdst.at[0]], sss[slot]).wait()

    for slot in range(_DEPTH):
        unpack_idx(slot, slot)
        issue(slot)

    def body(jj, carry):
        j0 = _DEPTH * jj
        for slot in range(_DEPTH):
            j = j0 + slot
            wait_gather(slot)

            @pl.when(jj > 0)
            def _():
                wait_scatter(slot)

            compute_chunk(slot)
            scatter(slot)
            unpack_idx(lax.rem(j + _DEPTH, _NCH), slot)
            issue(slot)
        return carry

    lax.fori_loop(0, _NCH // _DEPTH, body, 0)
    wait_gather(0)
    wait_scatter(0)
    compute_chunk(0)
    scatter(0)
    for slot in range(1, _DEPTH):
        wait_gather(slot)
        wait_scatter(slot)
    wait_scatter(0)
    plsc.subcore_barrier()

    pltpu.sync_copy(acc.at[pl.ds(s * _RPT, _RPT)],
                    out_hbm.at[pl.ds(c * _N + s * _RPT, _RPT)])


def _edge_pass(psrc, pdst, xl, xr, att_flat):
    mesh = plsc.VectorSubcoreMesh(core_axis_name="c", subcore_axis_name="s",
                                  num_cores=_NC, num_subcores=_NS)
    k = pl.kernel(
        _edge_body,
        out_type=jax.ShapeDtypeStruct((_NC * _N, _AW), jnp.float32),
        mesh=mesh,
        scratch_types=[
            pltpu.VMEM((_NPK, _CH), jnp.int32),
            pltpu.VMEM((_NPK, _CH), jnp.int32),
            pltpu.VMEM((_DEPTH, 16), jnp.int32),
            pltpu.VMEM((_DEPTH, 16), jnp.int32),
            pltpu.VMEM((_DEPTH, 16), jnp.int32),
            pltpu.VMEM((_CH, _D), jnp.float32),
            pltpu.VMEM((_CH, _D), jnp.float32),
            pltpu.VMEM((_CH, _D), jnp.float32),
            pltpu.VMEM((_CH, _D), jnp.float32),
            pltpu.VMEM((_CH, _D), jnp.float32),
            pltpu.VMEM((_CH, _D), jnp.float32),
            pltpu.VMEM((_CH, _D), jnp.float32),
            pltpu.VMEM((_CH, _D), jnp.float32),
            pltpu.VMEM((_CH, _AW), jnp.float32),
            pltpu.VMEM((_CH, _AW), jnp.float32),
            pltpu.VMEM((_CH, _AW), jnp.float32),
            pltpu.VMEM((_CH, _AW), jnp.float32),
            pltpu.VMEM((_D,), jnp.float32),
            pltpu.VMEM((_ZR, _AW), jnp.float32),
            pltpu.VMEM_SHARED((_N, _AW), jnp.float32),
            pltpu.SemaphoreType.DMA,
            pltpu.SemaphoreType.DMA,
            pltpu.SemaphoreType.DMA,
            pltpu.SemaphoreType.DMA,
            pltpu.SemaphoreType.DMA,
            pltpu.SemaphoreType.DMA,
            pltpu.SemaphoreType.DMA,
            pltpu.SemaphoreType.DMA,
            pltpu.SemaphoreType.DMA,
            pltpu.SemaphoreType.DMA,
            pltpu.SemaphoreType.DMA,
            pltpu.SemaphoreType.DMA,
        ],
        compiler_params=pltpu.CompilerParams(use_tc_tiling_on_sc=False,
                                             needs_layout_passes=False),
    )
    return k(psrc, pdst, xl, xr, att_flat)


def _final_body(a0_ref, a1_ref, x_ref, b_ref, g_ref, bt_ref, o_ref):
    a = a0_ref[...] + a1_ref[...]
    msg = a[:, :_D]
    den = a[:, _D:_D + _H]
    lane = lax.broadcasted_iota(jnp.int32, (_H, _D), 1) // (_D // _H)
    head = lax.broadcasted_iota(jnp.int32, (_H, _D), 0)
    expand = (lane == head).astype(jnp.float32)
    den_b = lax.dot_general(den, expand, (((1,), (0,)), ((), ())),
                            preferred_element_type=jnp.float32)
    o = msg / (den_b + 1e-16) + b_ref[...] + x_ref[...]
    m = jnp.mean(o, axis=1, keepdims=True)
    d = o - m
    var = jnp.mean(d * d, axis=1, keepdims=True)
    o = d * lax.rsqrt(var + 1e-5)
    o_ref[...] = o * g_ref[...] + bt_ref[...]


def _final(acc, x, bias, gamma, beta):
    blk = 1000
    return pl.pallas_call(
        _final_body,
        grid=(_N // blk,),
        in_specs=[
            pl.BlockSpec((blk, _AW), lambda i: (i, 0)),
            pl.BlockSpec((blk, _AW), lambda i: (_N // blk + i, 0)),
            pl.BlockSpec((blk, _D), lambda i: (i, 0)),
            pl.BlockSpec((1, _D), lambda i: (0, 0)),
            pl.BlockSpec((1, _D), lambda i: (0, 0)),
            pl.BlockSpec((1, _D), lambda i: (0, 0)),
        ],
        out_specs=pl.BlockSpec((blk, _D), lambda i: (i, 0)),
        out_shape=jax.ShapeDtypeStruct((_N, _D), jnp.float32),
    )(acc, acc, x, bias, gamma, beta)


def kernel(x, edge_index, W_l, W_r, att, bias, ln_gamma, ln_beta):
    src = edge_index[0].astype(jnp.int32).reshape(_NW, _EPW)
    dst = edge_index[1].astype(jnp.int32).reshape(_NW, _EPW)
    pad = 2 * _NPK * _CH - _EPW
    src = jnp.pad(src, ((0, 0), (0, pad))).reshape(_NW, _NPK, 2, _CH)
    dst = jnp.pad(dst, ((0, 0), (0, pad))).reshape(_NW, _NPK, 2, _CH)
    psrc = src[:, :, 0, :] | (src[:, :, 1, :] << 16)
    pdst = dst[:, :, 0, :] | (dst[:, :, 1, :] << 16)
    xl, xr = _project(x, W_l, W_r)
    acc = _edge_pass(psrc, pdst, xl, xr, att.reshape(_D))
    return _final(acc, x, bias[None, :], ln_gamma[None, :], ln_beta[None, :])
